# baseline (speedup 1.0000x reference)
import functools

import jax
import jax.numpy as jnp
from jax import lax
from jax.experimental import pallas as pl
from jax.experimental.pallas import tpu as pltpu
from jax.experimental.pallas import tpu_sc as plsc

_PAD_ROW = 5
_D = 128
_NC = 2
_NS = 16
_NW = _NC * _NS
_CHUNK = 128


def _body(idx_hbm, table_hbm, out_hbm, idx_v, rows_v, gsem, osem):
    c = lax.axis_index("c")
    s = lax.axis_index("s")
    wid = s * _NC + c
    n_chunks = idx_hbm.shape[0] // _NW
    base = wid * n_chunks

    pltpu.sync_copy(idx_hbm.at[pl.ds(base, n_chunks)], idx_v)

    def step(j, carry):
        buf = lax.rem(j, 2)
        gather = pltpu.make_async_copy(
            table_hbm.at[idx_v.at[j]], rows_v.at[buf], gsem)
        gather.start()
        gather.wait()
        out = pltpu.make_async_copy(
            rows_v.at[buf], out_hbm.at[pl.ds((base + j) * _CHUNK, _CHUNK)],
            osem)
        out.start()
        out.wait()
        return carry

    lax.fori_loop(0, n_chunks, step, 0)


def kernel(inputs, table):
    n_rows, n_cols = inputs.shape
    b_total = n_rows * n_cols
    table_masked = table.at[_PAD_ROW].set(0.0)
    idx = inputs.reshape(b_total // _CHUNK, _CHUNK).astype(jnp.int32)

    mesh = plsc.VectorSubcoreMesh(core_axis_name="c", subcore_axis_name="s")
    n_chunks = (b_total // _CHUNK) // _NW

    run = functools.partial(
        pl.kernel,
        out_type=jax.ShapeDtypeStruct((b_total, _D), jnp.float32),
        mesh=mesh,
        scratch_types=[
            pltpu.VMEM((n_chunks, _CHUNK), jnp.int32),
            pltpu.VMEM((2, _CHUNK, _D), jnp.float32),
            pltpu.SemaphoreType.DMA,
            pltpu.SemaphoreType.DMA,
        ],
    )(_body)

    out = run(idx, table_masked)
    return out.reshape(n_rows, n_cols, _D)


# speedup vs baseline: 1.0048x; 1.0048x over previous
import functools

import jax
import jax.numpy as jnp
from jax import lax
from jax.experimental import pallas as pl
from jax.experimental.pallas import tpu as pltpu
from jax.experimental.pallas import tpu_sc as plsc

_PAD_ROW = 5
_D = 128
_NC = 2
_NS = 16
_NW = _NC * _NS
_CHUNK = 128
_NBUF = 6
_G = 3
_W = 3


def _body(idx_hbm, table_hbm, out_hbm, idx_v, rows_v, gsem, osem):
    c = lax.axis_index("c")
    s = lax.axis_index("s")
    wid = s * _NC + c
    n = idx_hbm.shape[0] // _NW
    base = wid * n

    pltpu.sync_copy(idx_hbm.at[pl.ds(base, n)], idx_v)

    def start_gather(j):
        pltpu.make_async_copy(
            table_hbm.at[idx_v.at[j]], rows_v.at[lax.rem(j, _NBUF)],
            gsem).start()

    def wait_gather():
        pltpu.make_async_copy(
            table_hbm.at[idx_v.at[0]], rows_v.at[0], gsem).wait()

    def start_out(j):
        pltpu.make_async_copy(
            rows_v.at[lax.rem(j, _NBUF)],
            out_hbm.at[pl.ds((base + j) * _CHUNK, _CHUNK)], osem).start()

    def wait_out():
        pltpu.make_async_copy(
            rows_v.at[0], out_hbm.at[pl.ds(0, _CHUNK)], osem).wait()

    for j in range(_G):
        start_gather(j)

    def phase_a(j, carry):
        wait_gather()
        start_out(j)
        start_gather(j + _G)
        return carry

    lax.fori_loop(0, _W - 1, phase_a, 0)

    def phase_b(j, carry):
        wait_gather()
        start_out(j)
        wait_out()
        start_gather(j + _G)
        return carry

    lax.fori_loop(_W - 1, n - _G, phase_b, 0)

    def phase_c(j, carry):
        wait_gather()
        start_out(j)
        wait_out()
        return carry

    lax.fori_loop(n - _G, n, phase_c, 0)

    for _ in range(_W - 1):
        wait_out()


def kernel(inputs, table):
    n_rows, n_cols = inputs.shape
    b_total = n_rows * n_cols
    table_masked = table.at[_PAD_ROW].set(0.0)
    idx = inputs.reshape(b_total // _CHUNK, _CHUNK).astype(jnp.int32)

    mesh = plsc.VectorSubcoreMesh(core_axis_name="c", subcore_axis_name="s")
    n_chunks = (b_total // _CHUNK) // _NW

    run = functools.partial(
        pl.kernel,
        out_type=jax.ShapeDtypeStruct((b_total, _D), jnp.float32),
        mesh=mesh,
        scratch_types=[
            pltpu.VMEM((n_chunks, _CHUNK), jnp.int32),
            pltpu.VMEM((_NBUF, _CHUNK, _D), jnp.float32),
            pltpu.SemaphoreType.DMA,
            pltpu.SemaphoreType.DMA,
        ],
    )(_body)

    out = run(idx, table_masked)
    return out.reshape(n_rows, n_cols, _D)


# speedup vs baseline: 6.7926x; 6.7598x over previous
import functools

import jax
import jax.numpy as jnp
from jax import lax
from jax.experimental import pallas as pl
from jax.experimental.pallas import tpu as pltpu
from jax.experimental.pallas import tpu_sc as plsc

_PAD_ROW = 5
_D = 128
_NC = 2
_NS = 16
_NW = _NC * _NS
_CHUNK = 128
_NBUF = 3
_L = 16


def _body(idx_hbm, table_hbm, out_hbm, idx_v, table_v, obuf, isem, osem):
    c = lax.axis_index("c")
    s = lax.axis_index("s")
    wid = s * _NC + c
    n = idx_hbm.shape[0] // _NW
    base = wid * n

    pltpu.make_async_copy(idx_hbm.at[pl.ds(base, n)], idx_v, isem).start()
    pltpu.sync_copy(table_hbm, table_v)
    zero = jnp.zeros((_L,), jnp.float32)
    for jb in range(_D // _L):
        table_v[pl.ds(_PAD_ROW * _D + jb * _L, _L)] = zero
    pltpu.make_async_copy(idx_hbm.at[pl.ds(base, n)], idx_v, isem).wait()

    def build(j, buf):
        def row16(r16, carry):
            ids = idx_v[j, pl.ds(r16 * _L, _L)]
            for l in range(_L):
                off = ids[l] * _D
                rbase = (r16 * _L + l) * _D
                for jb in range(_D // _L):
                    obuf[buf, pl.ds(rbase + jb * _L, _L)] = (
                        table_v[pl.ds(off + jb * _L, _L)])
            return carry
        lax.fori_loop(0, _CHUNK // _L, row16, 0)

    def start_out(j):
        pltpu.make_async_copy(
            obuf.at[lax.rem(j, _NBUF)],
            out_hbm.at[pl.ds((base + j) * _CHUNK * _D, _CHUNK * _D)],
            osem).start()

    def wait_out():
        pltpu.make_async_copy(
            obuf.at[0], out_hbm.at[pl.ds(0, _CHUNK * _D)], osem).wait()

    for j in range(_NBUF - 1):
        build(j, j)
        start_out(j)

    def step(j, carry):
        wait_out()
        build(j, lax.rem(j, _NBUF))
        start_out(j)
        return carry

    lax.fori_loop(_NBUF - 1, n, step, 0)

    for _ in range(_NBUF - 1):
        wait_out()


def kernel(inputs, table):
    n_rows, n_cols = inputs.shape
    b_total = n_rows * n_cols
    idx = inputs.reshape(b_total // _CHUNK, _CHUNK).astype(jnp.int32)
    table_flat = table.reshape(-1)

    mesh = plsc.VectorSubcoreMesh(core_axis_name="c", subcore_axis_name="s")
    n_chunks = (b_total // _CHUNK) // _NW

    run = functools.partial(
        pl.kernel,
        out_type=jax.ShapeDtypeStruct((b_total * _D,), jnp.float32),
        mesh=mesh,
        scratch_types=[
            pltpu.VMEM((n_chunks, _CHUNK), jnp.int32),
            pltpu.VMEM((table_flat.shape[0],), jnp.float32),
            pltpu.VMEM((_NBUF, _CHUNK * _D), jnp.float32),
            pltpu.SemaphoreType.DMA,
            pltpu.SemaphoreType.DMA,
        ],
    )(_body)

    out = run(idx, table_flat)
    return out.reshape(n_rows, n_cols, _D)


# speedup vs baseline: 30.7269x; 4.5236x over previous
import functools

import jax
import jax.numpy as jnp
from jax import lax
from jax.experimental import pallas as pl
from jax.experimental.pallas import tpu as pltpu
from jax.experimental.pallas import tpu_sc as plsc

_PAD_ROW = 5
_D = 128
_NC = 2
_NS = 16
_NW = _NC * _NS
_CHUNK = 128
_NBUF = 4
_G = 2
_W = 2
_L = 16


def _body(idx_hbm, table_hbm, out_hbm, idx_v, table_v, table_sh, rows_v,
          isem, gsem, osem):
    c = lax.axis_index("c")
    s = lax.axis_index("s")
    wid = s * _NC + c
    n = idx_hbm.shape[0] // _NW
    base = wid * n

    pltpu.make_async_copy(idx_hbm.at[pl.ds(base, n)], idx_v, isem).start()

    @pl.when(s == 0)
    def _():
        pltpu.sync_copy(table_hbm, table_v)
        zero = jnp.zeros((_L,), jnp.float32)
        for jb in range(_D // _L):
            table_v[_PAD_ROW, pl.ds(jb * _L, _L)] = zero
        pltpu.sync_copy(table_v, table_sh)

    plsc.subcore_barrier()
    pltpu.make_async_copy(idx_hbm.at[pl.ds(base, n)], idx_v, isem).wait()

    def start_gather(j):
        pltpu.make_async_copy(
            table_sh.at[idx_v.at[j]], rows_v.at[lax.rem(j, _NBUF)],
            gsem).start()

    def wait_gather():
        pltpu.make_async_copy(
            table_sh.at[idx_v.at[0]], rows_v.at[0], gsem).wait()

    def start_out(j):
        pltpu.make_async_copy(
            rows_v.at[lax.rem(j, _NBUF)],
            out_hbm.at[pl.ds((base + j) * _CHUNK, _CHUNK)], osem).start()

    def wait_out():
        pltpu.make_async_copy(
            rows_v.at[0], out_hbm.at[pl.ds(0, _CHUNK)], osem).wait()

    for j in range(_G):
        start_gather(j)

    def phase_a(j, carry):
        wait_gather()
        start_out(j)
        start_gather(j + _G)
        return carry

    lax.fori_loop(0, _W - 1, phase_a, 0)

    def phase_b(j, carry):
        wait_gather()
        start_out(j)
        wait_out()
        start_gather(j + _G)
        return carry

    lax.fori_loop(_W - 1, n - _G, phase_b, 0)

    def phase_c(j, carry):
        wait_gather()
        start_out(j)
        wait_out()
        return carry

    lax.fori_loop(n - _G, n, phase_c, 0)

    for _ in range(_W - 1):
        wait_out()


def kernel(inputs, table):
    n_rows, n_cols = inputs.shape
    b_total = n_rows * n_cols
    idx = inputs.reshape(b_total // _CHUNK, _CHUNK).astype(jnp.int32)

    mesh = plsc.VectorSubcoreMesh(core_axis_name="c", subcore_axis_name="s")
    n_chunks = (b_total // _CHUNK) // _NW

    run = functools.partial(
        pl.kernel,
        out_type=jax.ShapeDtypeStruct((b_total, _D), jnp.float32),
        mesh=mesh,
        scratch_types=[
            pltpu.VMEM((n_chunks, _CHUNK), jnp.int32),
            pltpu.VMEM(table.shape, jnp.float32),
            pltpu.VMEM_SHARED(table.shape, jnp.float32),
            pltpu.VMEM((_NBUF, _CHUNK, _D), jnp.float32),
            pltpu.SemaphoreType.DMA,
            pltpu.SemaphoreType.DMA,
            pltpu.SemaphoreType.DMA,
        ],
    )(_body)

    out = run(idx, table)
    return out.reshape(n_rows, n_cols, _D)
